# baseline (speedup 1.0000x reference)
import functools

import jax
import jax.numpy as jnp
import numpy as np
from jax import lax
from jax.experimental import pallas as pl
from jax.experimental.pallas import tpu as pltpu
from jax.experimental.pallas import tpu_sc as plsc

NC, NS, L = 2, 16, 16
NW = NC * NS
B = 16384
QLEN = 20
NNEG = 50
NP = 52
D = 32
FACTOR = 0.5
BS = B // NW
BLK = 64
NBLK = BS // BLK
G = 4
QROWS = G * QLEN
NGRP = BLK // G
CROWS = 2 * NP
NCH = BLK // 2

TB = 1024
GRID = B // TB

def _lane():
    return lax.iota(jnp.int32, L)


def _hsum(v):
    lane = _lane()
    for k in (8, 4, 2, 1):
        v = v + jnp.take_along_axis(v, lane ^ k, axis=0)
    return v


def _bcast_lane(v, k):
    return jnp.take_along_axis(v, jnp.full((L,), k, jnp.int32), axis=0)


def _dots_rows(rows, rbase, alo, ahi, sc, orow, ocol):
    lane = _lane()
    zero = jnp.zeros((L,), jnp.float32)
    gid = lax.shift_right_logical(lane, 2)
    lid = lane & 3
    for grp in range(4):
        n0 = grp * L
        nn = min(L, NP - n0)
        vec = zero
        for j in range(4):
            quad = [n0 + j + 4 * r for r in range(4) if j + 4 * r < nn]
            if not quad:
                continue
            ss = []
            for n in quad:
                lo = rows[rbase + n, pl.ds(0, L)]
                hi = rows[rbase + n, pl.ds(L, L)]
                w = lo * alo + hi * ahi
                w = w + jnp.take_along_axis(w, lane ^ 8, axis=0)
                w = w + jnp.take_along_axis(w, lane ^ 4, axis=0)
                ss.append(w)
            m = ss[0]
            for r in range(1, len(ss)):
                m = jnp.where(gid == r, ss[r], m)
            m = m + jnp.take_along_axis(m, lane ^ 2, axis=0)
            m = m + jnp.take_along_axis(m, lane ^ 1, axis=0)
            vec = jnp.where(lid == j, m, vec)
        sc[orow, pl.ds(ocol + n0, L)] = vec


NBUF = 5


def _pipe4(nch, fire, wait, process, bufs):
    ahead = NBUF - 1
    for q in range(ahead):
        fire(q, *bufs[q])

    def rnd(co, carry):
        for q in range(NBUF):
            c = NBUF * co + q
            nb = bufs[(q + ahead) % NBUF]

            @pl.when(c + ahead < nch)
            def _(nb=nb, c=c):
                fire(c + ahead, *nb)

            wait(*bufs[q])
            process(c, bufs[q][0])
        return carry

    lax.fori_loop(0, nch // NBUF, rnd, 0)

    def tail(c, bq):
        wait(*bq)
        process(c, bq[0])

    for q in range(nch % NBUF):
        c = (nch // NBUF) * NBUF + q
        tail(c, bufs[q % NBUF])


@functools.cache
def _sc_scores_fn():
    @functools.partial(
        pl.kernel,
        mesh=plsc.VectorSubcoreMesh(core_axis_name="c", subcore_axis_name="s"),
        out_type=jax.ShapeDtypeStruct((2 * B, 128), jnp.float32),
        scratch_types=[
            pltpu.VMEM((BLK * NP,), jnp.int32),
            pltpu.VMEM((CROWS, D), jnp.float32),
            pltpu.VMEM((CROWS, D), jnp.float32),
            pltpu.VMEM((CROWS, D), jnp.float32),
            pltpu.VMEM((CROWS, D), jnp.float32),
            pltpu.VMEM((CROWS, D), jnp.float32),
            pltpu.VMEM((CROWS, D), jnp.float32),
            pltpu.VMEM((BLK,), jnp.int32),
            pltpu.VMEM((BLK, D), jnp.float32),
            pltpu.VMEM((BLK, D), jnp.float32),
            pltpu.VMEM((BLK, D), jnp.float32),
            pltpu.VMEM((BLK, D), jnp.float32),
            pltpu.VMEM((2 * BLK, 128), jnp.float32),
            pltpu.VMEM((D, D), jnp.float32),
            pltpu.VMEM((D,), jnp.float32),
            pltpu.SemaphoreType.DMA,
            pltpu.SemaphoreType.DMA,
            pltpu.SemaphoreType.DMA,
            pltpu.SemaphoreType.DMA,
            pltpu.SemaphoreType.DMA,
            pltpu.SemaphoreType.DMA,
        ],
        compiler_params=pltpu.CompilerParams(use_tc_tiling_on_sc=False),
    )
    def body(ent, wt, users, items, review, qidx, ni_p, nu_p, nw_p, wqt, bq,
             out,
             sidx, ra, rb, rc, rd, re_, rf, eidx, su, si, sw, sp, sc,
             swq, sbq, sa, sb, s3, s4, s5, s6):
        wid = lax.axis_index("s") * NC + lax.axis_index("c")
        pltpu.sync_copy(wqt, swq)
        pltpu.sync_copy(bq, sbq)
        zero = jnp.zeros((L,), jnp.float32)

        def blk_body(blk, carry0):
            base_e = wid * BS + blk * BLK

            def zrow(r, c):
                for c8 in range(8):
                    sc[r, pl.ds(c8 * L, L)] = zero
                return c
            lax.fori_loop(0, 2 * BLK, zrow, 0, unroll=4)

            for (src, tab, dst) in ((users, ent, su), (items, ent, si),
                                    (review, wt, sw)):
                pltpu.sync_copy(src.at[pl.ds(base_e, BLK)], eidx)
                pltpu.async_copy(tab.at[eidx], dst, sa).wait()

            qoff = pl.multiple_of(base_e * QLEN, 8)
            pltpu.sync_copy(qidx.at[pl.ds(qoff, BLK * QLEN)],
                            sidx.at[pl.ds(0, BLK * QLEN)])

            def q_fire(g, rbuf, sem):
                pltpu.async_copy(
                    wt.at[sidx.at[pl.ds(g * QROWS, QROWS)]],
                    rbuf.at[pl.ds(0, QROWS), :], sem)

            def q_wait(rbuf, sem):
                pltpu.make_async_copy(
                    wt.at[sidx.at[pl.ds(0, QROWS)]],
                    rbuf.at[pl.ds(0, QROWS), :], sem).wait()

            def q_process(g, rbuf):
                def ex_body(e4, carry):
                    ex = g * G + e4
                    rb0 = e4 * QLEN
                    qlo = zero
                    qhi = zero
                    vcnt = zero
                    for l in range(QLEN):
                        lo = rbuf[rb0 + l, pl.ds(0, L)]
                        hi = rbuf[rb0 + l, pl.ds(L, L)]
                        qlo = qlo + lo
                        qhi = qhi + hi
                        sab = _hsum(jnp.abs(lo) + jnp.abs(hi))
                        vcnt = vcnt + jnp.where(sab > 0.0, 1.0, 0.0)
                    scale = 1.0 / (vcnt + 1e-6)
                    qmlo = qlo * scale
                    qmhi = qhi * scale
                    qvlo = sbq[pl.ds(0, L)]
                    qvhi = sbq[pl.ds(L, L)]
                    for k in range(D):
                        qk = (_bcast_lane(qmlo, k) if k < L
                              else _bcast_lane(qmhi, k - L))
                        qvlo = qvlo + swq[k, pl.ds(0, L)] * qk
                        qvhi = qvhi + swq[k, pl.ds(L, L)] * qk
                    tlo = 2.0 / (1.0 + jnp.exp(-2.0 * qvlo)) - 1.0
                    thi = 2.0 / (1.0 + jnp.exp(-2.0 * qvhi)) - 1.0
                    ulo = su[ex, pl.ds(0, L)]
                    uhi = su[ex, pl.ds(L, L)]
                    plo = FACTOR * tlo + (1.0 - FACTOR) * ulo
                    phi = FACTOR * thi + (1.0 - FACTOR) * uhi
                    sp[ex, pl.ds(0, L)] = plo
                    sp[ex, pl.ds(L, L)] = phi
                    wlo = sw[ex, pl.ds(0, L)]
                    whi = sw[ex, pl.ds(L, L)]
                    ilo = si[ex, pl.ds(0, L)]
                    ihi = si[ex, pl.ds(L, L)]
                    p_uw = _hsum(ulo * wlo + uhi * whi)
                    p_iw = _hsum(ilo * wlo + ihi * whi)
                    p_s = _hsum(plo * ilo + phi * ihi)
                    lane = _lane()
                    pv = jnp.where(lane == 0, p_uw, zero)
                    pv = jnp.where(lane == 1, p_iw, pv)
                    pv = jnp.where(lane == 2, p_s, pv)
                    sc[2 * ex + 1, pl.ds(64, L)] = pv
                    return carry
                lax.fori_loop(0, G, ex_body, 0)

            _pipe4(NGRP, q_fire, q_wait, q_process,
                   ((ra, sa), (rb, sb), (rc, s3), (rd, s4), (re_, s5)))

            for (src, tab, anch, par, col) in (
                    (nu_p, wt, su, 0, 0),
                    (nw_p, wt, si, 0, 64),
                    (ni_p, ent, sp, 1, 0)):
                noff = pl.multiple_of(base_e * NP, 8)
                pltpu.sync_copy(src.at[pl.ds(noff, BLK * NP)],
                                sidx.at[pl.ds(0, BLK * NP)])

                def n_fire(c, rbuf, sem, tab=tab):
                    pltpu.async_copy(
                        tab.at[sidx.at[pl.ds(c * CROWS, CROWS)]], rbuf, sem)

                def n_wait(rbuf, sem, tab=tab):
                    pltpu.make_async_copy(
                        tab.at[sidx.at[pl.ds(0, CROWS)]], rbuf, sem).wait()

                def n_process(c, rbuf, anch=anch, par=par, col=col):
                    def e_body(e2, carry):
                        ex = c * 2 + e2
                        alo = anch[ex, pl.ds(0, L)]
                        ahi = anch[ex, pl.ds(L, L)]
                        _dots_rows(rbuf, e2 * NP, alo, ahi,
                                   sc, 2 * ex + par, col)
                        return carry
                    lax.fori_loop(0, 2, e_body, 0)

                _pipe4(NCH, n_fire, n_wait, n_process,
                       ((ra, sa), (rb, sb), (rc, s3), (rd, s4), (re_, s5)))

            pltpu.sync_copy(sc, out.at[pl.ds(2 * base_e, 2 * BLK)])
            return carry0

        lax.fori_loop(0, NBLK, blk_body, 0)

    return body


def _reduce_body(x_ref, m_ref, out_ref):
    i = pl.program_id(0)
    x = x_ref[...]
    m = m_ref[...]
    y = jax.nn.log_sigmoid(x * m) * jnp.abs(m)
    part = jnp.reshape(jnp.sum(y) * (-1.0 / B), (1, 1))

    @pl.when(i == 0)
    def _init():
        out_ref[...] = part

    @pl.when(i != 0)
    def _acc():
        out_ref[...] = out_ref[...] + part


_reduce = pl.pallas_call(
    _reduce_body,
    grid=(GRID,),
    in_specs=[
        pl.BlockSpec((2 * TB, 128), lambda i: (i, 0)),
        pl.BlockSpec((2 * TB, 128), lambda i: (i, 0)),
    ],
    out_specs=pl.BlockSpec((1, 1), lambda i: (0, 0)),
    out_shape=jax.ShapeDtypeStruct((1, 1), jnp.float32),
)


def _mask() -> jnp.ndarray:
    col = jnp.arange(128)
    even = jnp.where((col < NNEG) | ((col >= 64) & (col < 64 + NNEG)),
                     -1.0, 0.0)
    odd = jnp.where(col < NNEG, -1.0, 0.0)
    odd = jnp.where((col >= 64) & (col < 67), 1.0, odd)
    m2 = jnp.stack([even, odd]).astype(jnp.float32)
    return jnp.tile(m2, (B, 1))


def kernel(users, items, query_words, review_words, neg_items,
           neg_words_user, neg_words_item, word_table, entity_table, Wq, bq):
    ni_p = jnp.concatenate([neg_items, neg_items[:, :2]], axis=1).reshape(-1)
    nu_p = jnp.concatenate(
        [neg_words_user, neg_words_user[:, :2]], axis=1).reshape(-1)
    nw_p = jnp.concatenate(
        [neg_words_item, neg_words_item[:, :2]], axis=1).reshape(-1)
    scores = _sc_scores_fn()(
        entity_table, word_table, users, items, review_words,
        query_words.reshape(-1), ni_p, nu_p, nw_p, Wq.T, bq)
    out = _reduce(scores, _mask())
    return out.reshape(())


# speedup vs baseline: 1.0248x; 1.0248x over previous
import functools

import jax
import jax.numpy as jnp
import numpy as np
from jax import lax
from jax.experimental import pallas as pl
from jax.experimental.pallas import tpu as pltpu
from jax.experimental.pallas import tpu_sc as plsc

NC, NS, L = 2, 16, 16
NW = NC * NS
B = 16384
QLEN = 20
NNEG = 50
NP = 52
D = 32
FACTOR = 0.5
BS = B // NW
BLK = 64
NBLK = BS // BLK
G = 4
QROWS = G * QLEN
NGRP = BLK // G
CROWS = 2 * NP
NCH = BLK // 2

TB = 1024
GRID = B // TB

def _lane():
    return lax.iota(jnp.int32, L)


def _hsum(v):
    lane = _lane()
    for k in (8, 4, 2, 1):
        v = v + jnp.take_along_axis(v, lane ^ k, axis=0)
    return v


def _bcast_lane(v, k):
    return jnp.take_along_axis(v, jnp.full((L,), k, jnp.int32), axis=0)


def _dots_rows(rows, rbase, alo, ahi, sc, orow, ocol):
    lane = _lane()
    zero = jnp.zeros((L,), jnp.float32)
    gid = lax.shift_right_logical(lane, 2)
    lid = lane & 3
    for grp in range(4):
        n0 = grp * L
        nn = min(L, NP - n0)
        vec = zero
        for j in range(4):
            quad = [n0 + j + 4 * r for r in range(4) if j + 4 * r < nn]
            if not quad:
                continue
            ss = []
            for n in quad:
                lo = rows[rbase + n, pl.ds(0, L)]
                hi = rows[rbase + n, pl.ds(L, L)]
                w = lo * alo + hi * ahi
                w = w + jnp.take_along_axis(w, lane ^ 8, axis=0)
                w = w + jnp.take_along_axis(w, lane ^ 4, axis=0)
                ss.append(w)
            m = ss[0]
            for r in range(1, len(ss)):
                m = jnp.where(gid == r, ss[r], m)
            m = m + jnp.take_along_axis(m, lane ^ 2, axis=0)
            m = m + jnp.take_along_axis(m, lane ^ 1, axis=0)
            vec = jnp.where(lid == j, m, vec)
        sc[orow, pl.ds(ocol + n0, L)] = vec


NBUF = 4


def _pipe4(nch, fire, wait, process, bufs):
    ahead = NBUF - 1
    for q in range(ahead):
        fire(q, *bufs[q])

    def rnd(co, carry):
        for q in range(NBUF):
            c = NBUF * co + q
            nb = bufs[(q + ahead) % NBUF]

            @pl.when(c + ahead < nch)
            def _(nb=nb, c=c):
                fire(c + ahead, *nb)

            wait(*bufs[q])
            process(c, bufs[q][0])
        return carry

    lax.fori_loop(0, nch // NBUF, rnd, 0)

    def tail(c, bq):
        wait(*bq)
        process(c, bq[0])

    for q in range(nch % NBUF):
        c = (nch // NBUF) * NBUF + q
        tail(c, bufs[q % NBUF])


@functools.cache
def _sc_scores_fn():
    @functools.partial(
        pl.kernel,
        mesh=plsc.VectorSubcoreMesh(core_axis_name="c", subcore_axis_name="s"),
        out_type=jax.ShapeDtypeStruct((2 * B, 128), jnp.float32),
        scratch_types=[
            pltpu.VMEM((BLK * NP,), jnp.int32),
            pltpu.VMEM((CROWS, D), jnp.float32),
            pltpu.VMEM((CROWS, D), jnp.float32),
            pltpu.VMEM((CROWS, D), jnp.float32),
            pltpu.VMEM((CROWS, D), jnp.float32),
            pltpu.VMEM((BLK,), jnp.int32),
            pltpu.VMEM((BLK, D), jnp.float32),
            pltpu.VMEM((BLK, D), jnp.float32),
            pltpu.VMEM((BLK, D), jnp.float32),
            pltpu.VMEM((BLK, D), jnp.float32),
            pltpu.VMEM((2 * BLK, 128), jnp.float32),
            pltpu.VMEM((D, D), jnp.float32),
            pltpu.VMEM((D,), jnp.float32),
            pltpu.SemaphoreType.DMA,
            pltpu.SemaphoreType.DMA,
            pltpu.SemaphoreType.DMA,
            pltpu.SemaphoreType.DMA,
        ],
        compiler_params=pltpu.CompilerParams(use_tc_tiling_on_sc=False),
    )
    def body(ent, wt, users, items, review, qidx, ni_p, nu_p, nw_p, wqt, bq,
             out,
             sidx, ra, rb, rc, rd, eidx, su, si, sw, sp, sc,
             swq, sbq, sa, sb, s3, s4):
        wid = lax.axis_index("s") * NC + lax.axis_index("c")
        pltpu.sync_copy(wqt, swq)
        pltpu.sync_copy(bq, sbq)
        zero = jnp.zeros((L,), jnp.float32)

        def blk_body(blk, carry0):
            base_e = wid * BS + blk * BLK

            def zrow(r, c):
                for c8 in range(8):
                    sc[r, pl.ds(c8 * L, L)] = zero
                return c
            lax.fori_loop(0, 2 * BLK, zrow, 0, unroll=4)

            for (src, tab, dst) in ((users, ent, su), (items, ent, si),
                                    (review, wt, sw)):
                pltpu.sync_copy(src.at[pl.ds(base_e, BLK)], eidx)
                pltpu.async_copy(tab.at[eidx], dst, sa).wait()

            qoff = pl.multiple_of(base_e * QLEN, 8)
            pltpu.sync_copy(qidx.at[pl.ds(qoff, BLK * QLEN)],
                            sidx.at[pl.ds(0, BLK * QLEN)])

            def q_fire(g, rbuf, sem):
                pltpu.async_copy(
                    wt.at[sidx.at[pl.ds(g * QROWS, QROWS)]],
                    rbuf.at[pl.ds(0, QROWS), :], sem)

            def q_wait(rbuf, sem):
                pltpu.make_async_copy(
                    wt.at[sidx.at[pl.ds(0, QROWS)]],
                    rbuf.at[pl.ds(0, QROWS), :], sem).wait()

            def q_process(g, rbuf):
                def ex_body(e4, carry):
                    ex = g * G + e4
                    rb0 = e4 * QLEN
                    qlo = zero
                    qhi = zero
                    vcnt = zero
                    for l in range(QLEN):
                        lo = rbuf[rb0 + l, pl.ds(0, L)]
                        hi = rbuf[rb0 + l, pl.ds(L, L)]
                        qlo = qlo + lo
                        qhi = qhi + hi
                        sab = _hsum(jnp.abs(lo) + jnp.abs(hi))
                        vcnt = vcnt + jnp.where(sab > 0.0, 1.0, 0.0)
                    scale = 1.0 / (vcnt + 1e-6)
                    qmlo = qlo * scale
                    qmhi = qhi * scale
                    qvlo = sbq[pl.ds(0, L)]
                    qvhi = sbq[pl.ds(L, L)]
                    for k in range(D):
                        qk = (_bcast_lane(qmlo, k) if k < L
                              else _bcast_lane(qmhi, k - L))
                        qvlo = qvlo + swq[k, pl.ds(0, L)] * qk
                        qvhi = qvhi + swq[k, pl.ds(L, L)] * qk
                    tlo = 2.0 / (1.0 + jnp.exp(-2.0 * qvlo)) - 1.0
                    thi = 2.0 / (1.0 + jnp.exp(-2.0 * qvhi)) - 1.0
                    ulo = su[ex, pl.ds(0, L)]
                    uhi = su[ex, pl.ds(L, L)]
                    plo = FACTOR * tlo + (1.0 - FACTOR) * ulo
                    phi = FACTOR * thi + (1.0 - FACTOR) * uhi
                    sp[ex, pl.ds(0, L)] = plo
                    sp[ex, pl.ds(L, L)] = phi
                    wlo = sw[ex, pl.ds(0, L)]
                    whi = sw[ex, pl.ds(L, L)]
                    ilo = si[ex, pl.ds(0, L)]
                    ihi = si[ex, pl.ds(L, L)]
                    p_uw = _hsum(ulo * wlo + uhi * whi)
                    p_iw = _hsum(ilo * wlo + ihi * whi)
                    p_s = _hsum(plo * ilo + phi * ihi)
                    lane = _lane()
                    pv = jnp.where(lane == 0, p_uw, zero)
                    pv = jnp.where(lane == 1, p_iw, pv)
                    pv = jnp.where(lane == 2, p_s, pv)
                    sc[2 * ex + 1, pl.ds(64, L)] = pv
                    return carry
                lax.fori_loop(0, G, ex_body, 0)

            _pipe4(NGRP, q_fire, q_wait, q_process,
                   ((ra, sa), (rb, sb), (rc, s3), (rd, s4)))

            for (src, tab, anch, par, col) in (
                    (nu_p, wt, su, 0, 0),
                    (nw_p, wt, si, 0, 64),
                    (ni_p, ent, sp, 1, 0)):
                noff = pl.multiple_of(base_e * NP, 8)
                pltpu.sync_copy(src.at[pl.ds(noff, BLK * NP)],
                                sidx.at[pl.ds(0, BLK * NP)])

                def n_fire(c, rbuf, sem, tab=tab):
                    pltpu.async_copy(
                        tab.at[sidx.at[pl.ds(c * CROWS, CROWS)]], rbuf, sem)

                def n_wait(rbuf, sem, tab=tab):
                    pltpu.make_async_copy(
                        tab.at[sidx.at[pl.ds(0, CROWS)]], rbuf, sem).wait()

                def n_process(c, rbuf, anch=anch, par=par, col=col):
                    def e_body(e2, carry):
                        ex = c * 2 + e2
                        alo = anch[ex, pl.ds(0, L)]
                        ahi = anch[ex, pl.ds(L, L)]
                        _dots_rows(rbuf, e2 * NP, alo, ahi,
                                   sc, 2 * ex + par, col)
                        return carry
                    lax.fori_loop(0, 2, e_body, 0)

                _pipe4(NCH, n_fire, n_wait, n_process,
                       ((ra, sa), (rb, sb), (rc, s3), (rd, s4)))

            pltpu.sync_copy(sc, out.at[pl.ds(2 * base_e, 2 * BLK)])
            return carry0

        lax.fori_loop(0, NBLK, blk_body, 0)

    return body


def _reduce_body(x_ref, m_ref, out_ref):
    i = pl.program_id(0)
    x = x_ref[...]
    m = m_ref[...]
    y = jax.nn.log_sigmoid(x * m) * jnp.abs(m)
    part = jnp.reshape(jnp.sum(y) * (-1.0 / B), (1, 1))

    @pl.when(i == 0)
    def _init():
        out_ref[...] = part

    @pl.when(i != 0)
    def _acc():
        out_ref[...] = out_ref[...] + part


_reduce = pl.pallas_call(
    _reduce_body,
    grid=(GRID,),
    in_specs=[
        pl.BlockSpec((2 * TB, 128), lambda i: (i, 0)),
        pl.BlockSpec((2 * TB, 128), lambda i: (i, 0)),
    ],
    out_specs=pl.BlockSpec((1, 1), lambda i: (0, 0)),
    out_shape=jax.ShapeDtypeStruct((1, 1), jnp.float32),
)


def _mask() -> jnp.ndarray:
    col = jnp.arange(128)
    even = jnp.where((col < NNEG) | ((col >= 64) & (col < 64 + NNEG)),
                     -1.0, 0.0)
    odd = jnp.where(col < NNEG, -1.0, 0.0)
    odd = jnp.where((col >= 64) & (col < 67), 1.0, odd)
    m2 = jnp.stack([even, odd]).astype(jnp.float32)
    return jnp.tile(m2, (B, 1))


def kernel(users, items, query_words, review_words, neg_items,
           neg_words_user, neg_words_item, word_table, entity_table, Wq, bq):
    ni_p = jnp.concatenate([neg_items, neg_items[:, :2]], axis=1).reshape(-1)
    nu_p = jnp.concatenate(
        [neg_words_user, neg_words_user[:, :2]], axis=1).reshape(-1)
    nw_p = jnp.concatenate(
        [neg_words_item, neg_words_item[:, :2]], axis=1).reshape(-1)
    scores = _sc_scores_fn()(
        entity_table, word_table, users, items, review_words,
        query_words.reshape(-1), ni_p, nu_p, nw_p, Wq.T, bq)
    out = _reduce(scores, _mask())
    return out.reshape(())
